# Initial kernel scaffold; baseline (speedup 1.0000x reference)
import jax
import jax.numpy as jnp
from jax.experimental import pallas as pl


def kernel(cosine, label):
    raise NotImplementedError("write your pallas kernel here")


# speedup vs baseline: 3.1551x; 3.1551x over previous
import math

import jax
import jax.numpy as jnp
from jax.experimental import pallas as pl
from jax.experimental.pallas import tpu as pltpu

_SCALING = 30.0
_MARGIN = 0.5
_COS_M = math.cos(_MARGIN)
_SIN_M = math.sin(_MARGIN)
_THRESH = -math.cos(_MARGIN)
_MMV = math.sin(_MARGIN) * _MARGIN

_CB = 2048


def _fused_kernel(label_ref, cosine_ref, out_ref, acc_ref, tacc_ref, *, ncols):
    step = pl.program_id(0)
    nsteps = pl.num_programs(0)

    @pl.when(step == 0)
    def _init():
        acc_ref[...] = jnp.zeros_like(acc_ref)
        tacc_ref[...] = jnp.zeros_like(tacc_ref)

    x = cosine_ref[...]
    col = step * _CB + jax.lax.broadcasted_iota(jnp.int32, x.shape, 1)
    valid = col < ncols
    e = jnp.where(valid, jnp.exp(x * _SCALING - _SCALING), 0.0)
    acc_ref[...] += jnp.sum(e, axis=1, keepdims=True)
    hit = col == label_ref[...]
    tacc_ref[...] += jnp.sum(jnp.where(hit, x, 0.0), axis=1, keepdims=True)

    @pl.when(step == nsteps - 1)
    def _fin():
        s = acc_ref[...]
        t = tacc_ref[...]
        tc = jnp.clip(t, -1.0, 1.0)
        tr = jnp.where(
            t > _THRESH,
            tc * _COS_M - jnp.sqrt(jnp.maximum(1.0 - tc * tc, 0.0)) * _SIN_M,
            t - _MMV,
        )
        s2 = s - jnp.exp(t * _SCALING - _SCALING) + jnp.exp(tr * _SCALING - _SCALING)
        ce = jnp.log(s2) - (tr * _SCALING - _SCALING)
        p = jnp.exp(-ce)
        loss = (1.0 - p) * ce
        out_ref[...] = jnp.sum(loss, keepdims=True) / loss.shape[0]


def kernel(cosine, label):
    b, c = cosine.shape
    label2d = label.astype(jnp.int32).reshape(b, 1)
    nsteps = (c + _CB - 1) // _CB
    import functools

    out = pl.pallas_call(
        functools.partial(_fused_kernel, ncols=c),
        grid=(nsteps,),
        in_specs=[
            pl.BlockSpec((b, 1), lambda i: (0, 0)),
            pl.BlockSpec((b, _CB), lambda i: (0, i)),
        ],
        out_specs=pl.BlockSpec((1, 1), lambda i: (0, 0)),
        out_shape=jax.ShapeDtypeStruct((1, 1), jnp.float32),
        scratch_shapes=[
            pltpu.VMEM((b, 1), jnp.float32),
            pltpu.VMEM((b, 1), jnp.float32),
        ],
    )(label2d, cosine)
    return out[0, 0]
